# baseline (speedup 1.0000x reference)
import jax
import jax.numpy as jnp
from jax import lax
from jax.experimental import pallas as pl
from jax.experimental.pallas import tpu as pltpu

_T, _D, _E, _F = 2048, 768, 8, 768
_BT = 512
_NSTEP = _T // _BT


def _moe_body(x_ref, wg_ref, bg_ref, w1_ref, b1_ref, w2_ref, b2_ref,
              out_ref, hw_ref):
    logits = (
        jnp.dot(x_ref[...], wg_ref[...], preferred_element_type=jnp.float32)
        + bg_ref[...]
    )
    ii = lax.broadcasted_iota(jnp.int32, logits.shape, 1)
    m1 = jnp.max(logits, axis=1, keepdims=True)
    i1 = jnp.min(jnp.where(logits == m1, ii, _E), axis=1, keepdims=True)
    masked = jnp.where(ii == i1, -jnp.inf, logits)
    m2 = jnp.max(masked, axis=1, keepdims=True)
    i2 = jnp.min(jnp.where(masked == m2, ii, _E), axis=1, keepdims=True)
    e2 = jnp.exp(m2 - m1)
    denom = 1.0 + e2
    g1 = 1.0 / denom
    g2 = e2 / denom

    x = x_ref[...]
    for e in range(_E):
        sl = pl.ds(e * _F, _F)
        he = jnp.maximum(
            jnp.dot(x, w1_ref[:, sl], preferred_element_type=jnp.float32)
            + b1_ref[:, sl],
            0.0,
        )
        we = jnp.where(i1 == e, g1, 0.0) + jnp.where(i2 == e, g2, 0.0)
        hw_ref[:, sl] = he * we

    w8 = jnp.where(ii == i1, g1, 0.0) + jnp.where(ii == i2, g2, 0.0)
    out_ref[...] = (
        jnp.dot(hw_ref[...], w2_ref[...], preferred_element_type=jnp.float32)
        + jnp.dot(w8, b2_ref[...], preferred_element_type=jnp.float32)
    )


def kernel(moe_inp, Wg, bg, W1, b1, W2, b2):
    W1all = W1.transpose(1, 0, 2).reshape(_D, _E * _F)
    W2all = W2.reshape(_E * _F, _D)
    return pl.pallas_call(
        _moe_body,
        grid=(_NSTEP,),
        in_specs=[
            pl.BlockSpec((_BT, _D), lambda b: (b, 0)),
            pl.BlockSpec((_D, _E), lambda b: (0, 0)),
            pl.BlockSpec((1, _E), lambda b: (0, 0)),
            pl.BlockSpec((_D, _E * _F), lambda b: (0, 0)),
            pl.BlockSpec((1, _E * _F), lambda b: (0, 0)),
            pl.BlockSpec((_E * _F, _D), lambda b: (0, 0)),
            pl.BlockSpec((_E, _D), lambda b: (0, 0)),
        ],
        out_specs=pl.BlockSpec((_BT, _D), lambda b: (b, 0)),
        out_shape=jax.ShapeDtypeStruct((_T, _D), jnp.float32),
        scratch_shapes=[pltpu.VMEM((_BT, _E * _F), jnp.float32)],
        compiler_params=pltpu.CompilerParams(
            dimension_semantics=("arbitrary",),
        ),
    )(moe_inp, Wg, bg.reshape(1, _E), W1all, b1.reshape(1, _E * _F),
      W2all, b2)


# speedup vs baseline: 1.6936x; 1.6936x over previous
import jax
import jax.numpy as jnp
from jax.experimental import pallas as pl
from jax.experimental.pallas import tpu as pltpu

_T, _D, _E, _F = 2048, 768, 8, 768


def _moe_body(x_ref, wg_ref, bg_ref, w1_ref, b1_ref, w2_ref, b2_ref,
              out_ref, logits_ref):
    e = pl.program_id(0)

    @pl.when(e == 0)
    def _():
        logits_ref[...] = (
            jnp.dot(x_ref[...], wg_ref[...], preferred_element_type=jnp.float32)
            + bg_ref[...]
        )

    logits = logits_ref[...]
    ii = jax.lax.broadcasted_iota(jnp.int32, logits.shape, 1)
    m1 = jnp.max(logits, axis=1, keepdims=True)
    i1 = jnp.min(jnp.where(logits == m1, ii, _E), axis=1, keepdims=True)
    masked = jnp.where(ii == i1, -jnp.inf, logits)
    m2 = jnp.max(masked, axis=1, keepdims=True)
    i2 = jnp.min(jnp.where(masked == m2, ii, _E), axis=1, keepdims=True)
    e2 = jnp.exp(m2 - m1)
    denom = 1.0 + e2
    w = jnp.where(i1 == e, 1.0 / denom, 0.0) + jnp.where(i2 == e, e2 / denom, 0.0)

    h = jnp.maximum(
        jnp.dot(x_ref[...], w1_ref[0], preferred_element_type=jnp.float32) + b1_ref[0],
        0.0,
    )
    y = jnp.dot(h, w2_ref[0], preferred_element_type=jnp.float32) + b2_ref[0]

    @pl.when(e == 0)
    def _():
        out_ref[...] = w * y

    @pl.when(e > 0)
    def _():
        out_ref[...] += w * y


def kernel(moe_inp, Wg, bg, W1, b1, W2, b2):
    return pl.pallas_call(
        _moe_body,
        grid=(_E,),
        in_specs=[
            pl.BlockSpec((_T, _D), lambda e: (0, 0)),
            pl.BlockSpec((_D, _E), lambda e: (0, 0)),
            pl.BlockSpec((1, _E), lambda e: (0, 0)),
            pl.BlockSpec((1, _D, _F), lambda e: (e, 0, 0)),
            pl.BlockSpec((1, 1, _F), lambda e: (e, 0, 0)),
            pl.BlockSpec((1, _F, _D), lambda e: (e, 0, 0)),
            pl.BlockSpec((1, 1, _D), lambda e: (e, 0, 0)),
        ],
        out_specs=pl.BlockSpec((_T, _D), lambda e: (0, 0)),
        out_shape=jax.ShapeDtypeStruct((_T, _D), jnp.float32),
        scratch_shapes=[pltpu.VMEM((_T, _E), jnp.float32)],
        compiler_params=pltpu.CompilerParams(
            dimension_semantics=("arbitrary",),
        ),
    )(moe_inp, Wg, bg.reshape(1, _E), W1, b1.reshape(_E, 1, _F), W2,
      b2.reshape(_E, 1, _D))
